# baseline (speedup 1.0000x reference)
import functools

import jax
import jax.numpy as jnp
from jax import lax
from jax.experimental import pallas as pl
from jax.experimental.pallas import tpu as pltpu
from jax.experimental.pallas import tpu_sc as plsc

NODES = 10000
NODES_PAD = 10240
EDGES = 320000
IN_C = 128
OUT_C = 128
NC = 2
NS = 16
NTILES = NC * NS
EPT = EDGES // NTILES
LANES = 16
WIN = 10112
SLICE = NODES_PAD // NS

_mesh = plsc.VectorSubcoreMesh(core_axis_name="c", subcore_axis_name="s")
_sc_params = pltpu.CompilerParams(needs_layout_passes=False)


def _zero(ref, n):
    zeros = jnp.zeros((LANES,), jnp.float32)

    @plsc.parallel_loop(0, n, step=LANES, unroll=4)
    def _(s):
        ref[pl.ds(s, LANES)] = zeros


def _rsqrt_newton(d):
    i = plsc.bitcast(d, jnp.int32)
    i = jnp.int32(0x5F3759DF) - lax.shift_right_logical(i, 1)
    y = plsc.bitcast(i, jnp.float32)
    for _ in range(3):
        y = y * (1.5 - 0.5 * d * y * y)
    return y


@functools.partial(
    pl.kernel,
    out_type=[
        jax.ShapeDtypeStruct((NC * NODES_PAD,), jnp.float32),
        jax.ShapeDtypeStruct((NC * NODES_PAD,), jnp.float32),
        jax.ShapeDtypeStruct((NC * NODES_PAD,), jnp.float32),
    ],
    mesh=_mesh,
    compiler_params=_sc_params,
    scratch_types=[
        pltpu.VMEM((2, WIN), jnp.int32),
        pltpu.VMEM((EPT,), jnp.float32),
        pltpu.VMEM((NODES_PAD,), jnp.float32),
        pltpu.VMEM((NODES_PAD,), jnp.float32),
        pltpu.VMEM((NODES_PAD,), jnp.float32),
        pltpu.VMEM((NS * SLICE,), jnp.float32),
        pltpu.VMEM((SLICE,), jnp.float32),
        pltpu.VMEM((SLICE,), jnp.float32),
        pltpu.VMEM_SHARED((NS * NODES_PAD,), jnp.float32),
        pltpu.VMEM_SHARED((NS * NODES_PAD,), jnp.float32),
        pltpu.VMEM_SHARED((NODES_PAD,), jnp.float32),
        pltpu.SemaphoreType.DMA,
    ],
)
def _sc_main(ei_hbm, ew_hbm, a_out, c_out, inv_out,
             rc_v, ew_v, acc_a, acc_c, dis_v, red_v, sl_a, sl_inv,
             sh_a, sh_c, sh_dis, sem):
    cid = lax.axis_index("c")
    sid = lax.axis_index("s")
    wid = sid * NC + cid
    nbase = sid * SLICE

    def _stage_slices(sh):
        copies = [
            pltpu.async_copy(
                sh.at[pl.ds(k * NODES_PAD + nbase, SLICE)],
                red_v.at[pl.ds(k * SLICE, SLICE)], sem)
            for k in range(NS)
        ]
        for cp in copies:
            cp.wait()

    def _reduce_slices(dst):
        @plsc.parallel_loop(0, SLICE, step=LANES, unroll=4)
        def _(s):
            tot = red_v[pl.ds(s, LANES)]
            for k in range(1, NS):
                tot = tot + red_v[pl.ds(k * SLICE + s, LANES)]
            dst[pl.ds(s, LANES)] = tot

    _zero(acc_a, NODES_PAD)
    for sub in range(2):
        base = (sid * 2 + sub) * EPT
        win = (base // 128) * 128
        off = base - win
        pltpu.sync_copy(ei_hbm.at[:, pl.ds(win, WIN)], rc_v)
        pltpu.sync_copy(ew_hbm.at[pl.ds(base, EPT)], ew_v)

        @plsc.parallel_loop(0, EPT, step=LANES, unroll=2)
        def _(s):
            idx = rc_v[1, pl.ds(off + s, LANES)]
            w = ew_v[pl.ds(s, LANES)]
            plsc.addupdate_scatter(acc_a, [idx], w)

    pltpu.sync_copy(acc_a, sh_a.at[pl.ds(sid * NODES_PAD, NODES_PAD)])
    plsc.subcore_barrier()

    _stage_slices(sh_a)
    _reduce_slices(sl_a)

    @plsc.parallel_loop(0, SLICE, step=LANES, unroll=4)
    def _(s):
        deg = sl_a[pl.ds(s, LANES)] + 1.0
        pos = deg > 0.0
        dis = jnp.where(pos, _rsqrt_newton(deg), 0.0)
        inv = jnp.where(pos, 1.0 / deg, 0.0)
        sl_a[pl.ds(s, LANES)] = dis
        sl_inv[pl.ds(s, LANES)] = inv

    pltpu.sync_copy(sl_a, sh_dis.at[pl.ds(nbase, SLICE)])
    pltpu.sync_copy(sl_inv, inv_out.at[pl.ds(cid * NODES_PAD + nbase, SLICE)])
    plsc.subcore_barrier()

    pltpu.sync_copy(sh_dis, dis_v)
    base = wid * EPT
    win = (base // 128) * 128
    off = base - win
    pltpu.sync_copy(ei_hbm.at[:, pl.ds(win, WIN)], rc_v)
    pltpu.sync_copy(ew_hbm.at[pl.ds(base, EPT)], ew_v)
    _zero(acc_a, NODES_PAD)
    _zero(acc_c, NODES_PAD)

    @plsc.parallel_loop(0, EPT, step=LANES, unroll=2)
    def _(s):
        r = rc_v[0, pl.ds(off + s, LANES)]
        c = rc_v[1, pl.ds(off + s, LANES)]
        w = ew_v[pl.ds(s, LANES)]
        dr = plsc.load_gather(dis_v, [r])
        dc = plsc.load_gather(dis_v, [c])
        nrm = dr * w * dc
        plsc.addupdate_scatter(acc_a, [c], nrm)
        plsc.addupdate_scatter(acc_c, [r], nrm)

    pltpu.sync_copy(acc_a, sh_a.at[pl.ds(sid * NODES_PAD, NODES_PAD)])
    pltpu.sync_copy(acc_c, sh_c.at[pl.ds(sid * NODES_PAD, NODES_PAD)])
    plsc.subcore_barrier()

    _stage_slices(sh_a)
    _reduce_slices(sl_a)
    pltpu.sync_copy(sl_a, a_out.at[pl.ds(cid * NODES_PAD + nbase, SLICE)])
    _stage_slices(sh_c)
    _reduce_slices(sl_a)
    pltpu.sync_copy(sl_a, c_out.at[pl.ds(cid * NODES_PAD + nbase, SLICE)])


def _tc_tail_body(a_ref, c_ref, inv_ref, w1_ref, b1_ref, w2_ref, b2_ref,
                  out_ref):
    def halves(ref):
        lo = jnp.reshape(ref[pl.ds(0, NODES)], (1, NODES))
        hi = jnp.reshape(ref[pl.ds(NODES_PAD, NODES)], (1, NODES))
        return lo, hi

    inv = halves(inv_ref)[0]
    a0, a1 = halves(a_ref)
    c0, c1 = halves(c_ref)
    a = a0 + a1 + inv
    c = c0 + c1 + inv
    ones = jnp.ones((IN_C, 1), jnp.float32)
    s1 = lax.dot_general(w1_ref[...], ones, (((0,), (0,)), ((), ())))
    h1 = jnp.maximum(s1 * a + b1_ref[...], 0.0)
    v = jnp.sum(h1 * c, axis=1, keepdims=True)
    out = lax.dot_general(v, w2_ref[...], (((0,), (0,)), ((), ())))
    out_ref[...] = out * (1.0 / NODES) + b2_ref[...]


_tc_tail = pl.pallas_call(
    _tc_tail_body,
    out_shape=jax.ShapeDtypeStruct((1, OUT_C), jnp.float32),
)


def kernel(x, edge_index, edge_attr, W1, b1, W2, b2):
    del x
    a2, c2, inv2 = _sc_main(edge_index, edge_attr)
    return _tc_tail(a2, c2, inv2, W1, jnp.reshape(b1, (IN_C, 1)),
                    W2, jnp.reshape(b2, (1, OUT_C)))


# speedup vs baseline: 1.0823x; 1.0823x over previous
import functools

import jax
import jax.numpy as jnp
from jax import lax
from jax.experimental import pallas as pl
from jax.experimental.pallas import tpu as pltpu
from jax.experimental.pallas import tpu_sc as plsc

NODES = 10000
EDGES = 320000
IN_C = 128
OUT_C = 128
NC = 2
NS = 16
NTILES = NC * NS
EPT = EDGES // NTILES
LANES = 16
VECS_E = EPT // LANES
VECS_N = NODES // LANES
WIN = 10112

_mesh = plsc.VectorSubcoreMesh(core_axis_name="c", subcore_axis_name="s")
_sc_params = pltpu.CompilerParams(needs_layout_passes=False)


def _wid():
    return lax.axis_index("s") * NC + lax.axis_index("c")


def _zero(ref):
    zeros = jnp.zeros((LANES,), jnp.float32)

    @plsc.parallel_loop(0, NODES, step=LANES, unroll=8)
    def _(s):
        ref[pl.ds(s, LANES)] = zeros


@functools.partial(
    pl.kernel,
    out_type=jax.ShapeDtypeStruct((NTILES, NODES), jnp.float32),
    mesh=_mesh,
    compiler_params=_sc_params,
    scratch_types=[
        pltpu.VMEM((2, WIN), jnp.int32),
        pltpu.VMEM((EPT,), jnp.float32),
        pltpu.VMEM((NODES,), jnp.float32),
    ],
)
def _sc_deg(ei_hbm, ew_hbm, out_hbm, rc_v, ew_v, acc_v):
    base = _wid() * EPT
    win = (base // 128) * 128
    off = base - win
    pltpu.sync_copy(ei_hbm.at[:, pl.ds(win, WIN)], rc_v)
    pltpu.sync_copy(ew_hbm.at[pl.ds(base, EPT)], ew_v)
    _zero(acc_v)

    @plsc.parallel_loop(0, EPT, step=LANES, unroll=8)
    def _(s):
        idx = rc_v[1, pl.ds(off + s, LANES)]
        w = ew_v[pl.ds(s, LANES)]
        plsc.addupdate_scatter(acc_v, [idx], w)
    pltpu.sync_copy(acc_v, out_hbm.at[_wid()])


def _tc_norm_body(degp_ref, dis_ref, inv_ref):
    deg = jnp.sum(degp_ref[...], axis=0, keepdims=True) + 1.0
    pos = deg > 0
    dis_ref[...] = jnp.where(pos, lax.rsqrt(deg), 0.0)
    inv_ref[...] = jnp.where(pos, 1.0 / deg, 0.0)


_tc_norm = pl.pallas_call(
    _tc_norm_body,
    out_shape=[
        jax.ShapeDtypeStruct((1, NODES), jnp.float32),
        jax.ShapeDtypeStruct((1, NODES), jnp.float32),
    ],
)


@functools.partial(
    pl.kernel,
    out_type=[
        jax.ShapeDtypeStruct((NTILES, NODES), jnp.float32),
        jax.ShapeDtypeStruct((NTILES, NODES), jnp.float32),
    ],
    mesh=_mesh,
    compiler_params=_sc_params,
    scratch_types=[
        pltpu.VMEM((2, WIN), jnp.int32),
        pltpu.VMEM((EPT,), jnp.float32),
        pltpu.VMEM((NODES,), jnp.float32),
        pltpu.VMEM((NODES,), jnp.float32),
        pltpu.VMEM((NODES,), jnp.float32),
    ],
)
def _sc_edges(ei_hbm, ew_hbm, dis_hbm, out_a, out_c,
              rc_v, ew_v, dis_v, acc_a, acc_c):
    base = _wid() * EPT
    win = (base // 128) * 128
    off = base - win
    pltpu.sync_copy(dis_hbm.at[0], dis_v)
    pltpu.sync_copy(ei_hbm.at[:, pl.ds(win, WIN)], rc_v)
    pltpu.sync_copy(ew_hbm.at[pl.ds(base, EPT)], ew_v)
    _zero(acc_a)
    _zero(acc_c)

    @plsc.parallel_loop(0, EPT, step=LANES, unroll=8)
    def _(s):
        r = rc_v[0, pl.ds(off + s, LANES)]
        c = rc_v[1, pl.ds(off + s, LANES)]
        w = ew_v[pl.ds(s, LANES)]
        dr = plsc.load_gather(dis_v, [r])
        dc = plsc.load_gather(dis_v, [c])
        nrm = dr * w * dc
        plsc.addupdate_scatter(acc_a, [c], nrm)
        plsc.addupdate_scatter(acc_c, [r], nrm)
    pltpu.sync_copy(acc_a, out_a.at[_wid()])
    pltpu.sync_copy(acc_c, out_c.at[_wid()])


def _tc_tail_body(ap_ref, cp_ref, inv_ref, w1_ref, b1_ref, w2_ref, b2_ref,
                  out_ref):
    inv = inv_ref[...]
    a = jnp.sum(ap_ref[...], axis=0, keepdims=True) + inv
    c = jnp.sum(cp_ref[...], axis=0, keepdims=True) + inv
    ones = jnp.ones((IN_C, 1), jnp.float32)
    s1 = lax.dot_general(w1_ref[...], ones, (((0,), (0,)), ((), ())))
    h1 = jnp.maximum(s1 * a + b1_ref[...], 0.0)
    v = jnp.sum(h1 * c, axis=1, keepdims=True)
    out = lax.dot_general(v, w2_ref[...], (((0,), (0,)), ((), ())))
    out_ref[...] = out * (1.0 / NODES) + b2_ref[...]


_tc_tail = pl.pallas_call(
    _tc_tail_body,
    out_shape=jax.ShapeDtypeStruct((1, OUT_C), jnp.float32),
)


def kernel(x, edge_index, edge_attr, W1, b1, W2, b2):
    del x
    deg_part = _sc_deg(edge_index, edge_attr)
    dis, inv = _tc_norm(deg_part)
    a_part, c_part = _sc_edges(edge_index, edge_attr, dis)
    return _tc_tail(a_part, c_part, inv, W1, jnp.reshape(b1, (IN_C, 1)),
                    W2, jnp.reshape(b2, (1, OUT_C)))
